# Initial kernel scaffold; baseline (speedup 1.0000x reference)
import jax
import jax.numpy as jnp
from jax.experimental import pallas as pl


def kernel(x, table):
    raise NotImplementedError("write your pallas kernel here")


# speedup vs baseline: 1.9598x; 1.9598x over previous
import functools

import jax
import jax.numpy as jnp
from jax import lax
from jax.experimental import pallas as pl
from jax.experimental.pallas import tpu as pltpu
import jax.experimental.pallas.tpu_sc as plsc

_NC = 2
_NS = 16
_NW = _NC * _NS

_CH = 4
_NBUF = 2


@functools.partial(jax.jit, static_argnums=())
def _sc_gather(table, idx3):
  nw, nch, ch = idx3.shape
  d = table.shape[1]
  b_total = nw * nch * ch
  mesh = plsc.VectorSubcoreMesh(core_axis_name="c", subcore_axis_name="s")

  @functools.partial(
      pl.kernel,
      out_type=jax.ShapeDtypeStruct((b_total, d), jnp.float32),
      mesh=mesh,
      scratch_types=[
          pltpu.VMEM((nch, ch), jnp.int32),
          *[pltpu.VMEM((ch, d), jnp.float32) for _ in range(_NBUF)],
          *[pltpu.SemaphoreType.DMA for _ in range(2 * _NBUF)],
      ],
  )
  def k(table_hbm, idx_hbm, out_hbm, idx_v, *rest):
    bufs = rest[:_NBUF]
    gsems = rest[_NBUF:2 * _NBUF]
    ssems = rest[2 * _NBUF:]
    wid = lax.axis_index("s") * _NC + lax.axis_index("c")
    base_row = wid * (nch * ch)

    pltpu.sync_copy(idx_hbm.at[wid], idx_v)

    def gather_start(b, g):
      pltpu.async_copy(table_hbm.at[idx_v.at[g]], bufs[b], gsems[b])

    def gather_wait(b):
      pltpu.make_async_copy(table_hbm.at[idx_v.at[0]], bufs[b],
                            gsems[b]).wait()

    def scatter_start(b, g):
      pltpu.async_copy(bufs[b], out_hbm.at[pl.ds(base_row + g * ch, ch)],
                       ssems[b])

    def scatter_wait(b):
      pltpu.make_async_copy(bufs[b], out_hbm.at[pl.ds(0, ch)],
                            ssems[b]).wait()

    for b in range(_NBUF):
      gather_start(b, b)

    @pl.loop(0, nch // _NBUF)
    def _(o):
      for b in range(_NBUF):
        g = o * _NBUF + b
        gather_wait(b)
        scatter_start(b, g)
        scatter_wait(b)
        nxt = g + _NBUF

        @pl.when(nxt < nch)
        def _():
          gather_start(b, nxt)

  return k(table, idx3)


def kernel(x, table):
  b, t = x.shape
  vocab = table.shape[0]
  idx = x.reshape(-1).astype(jnp.int32)
  b_total = idx.shape[0]
  r = b_total // _NW
  idx3 = idx.reshape(_NW, r // _CH, _CH)
  out = _sc_gather(table, idx3)
  return out.reshape(b, t, vocab)
